# baseline (speedup 1.0000x reference)
import dataclasses
import functools
import math

import jax
import jax.numpy as jnp
from jax import lax
from jax.experimental import pallas as pl
from jax.experimental.pallas import tpu as pltpu
from jax.experimental.pallas import tpu_sc as plsc

E = 8
K = 2
H = 1024
I = 2048
T = 2048
CF = 2.0
C = int(math.ceil(CF * T * K / E))
A = T * K

NC, NS, L = 2, 16, 16
NW = NC * NS
APW = A // NW
TPW = T // NW
NSL = APW // L
ROWCH = 32
NCH = APW // ROWCH

_MESH = plsc.VectorSubcoreMesh(core_axis_name="c", subcore_axis_name="s",
                               num_cores=NC, num_subcores=NS)

_SC_PARAMS = pltpu.CompilerParams()
if "needs_layout_passes" in pltpu.CompilerParams.__dataclass_fields__:
    _SC_PARAMS = dataclasses.replace(_SC_PARAMS, needs_layout_passes=False)


def _routing_body(ef_hbm, x_hbm, aff_hbm, buf_hbm, dst_hbm, w_hbm,
                  ef_v, hist, aff_v, dstc, tokc, kbuf, wbuf, rows0, rows1,
                  g0, g1, s0, s1):
    wid = lax.axis_index("s") * NC + lax.axis_index("c")
    base = wid * APW
    tok0 = wid * TPW

    pltpu.sync_copy(ef_hbm, ef_v)
    pltpu.sync_copy(aff_hbm.at[pl.ds(tok0, TPW)], aff_v)

    lanes = lax.iota(jnp.int32, L)
    ones = jnp.ones((L,), jnp.int32)
    hist[...] = jnp.zeros((L,), jnp.int32)

    @pl.loop(0, wid * NSL)
    def _(s):
        v = ef_v[pl.ds(s * L, L)]
        plsc.addupdate_scatter(hist, [v], ones)

    for s in range(NSL):
        v = ef_v[pl.ds(base + s * L, L)]
        rank = jnp.zeros((L,), jnp.int32)
        for e in range(E):
            m = v == e
            cs = plsc.cumsum(jnp.where(m, 1, 0).astype(jnp.int32))
            rank = jnp.where(m, cs - 1, rank)
        basecnt = plsc.load_gather(hist, [v])
        plsc.addupdate_scatter(hist, [v], ones)
        pos = basecnt + rank
        keep = pos < C
        posc = jnp.minimum(pos, C - 1)
        src = v * C + posc
        dstc[s // 2, pl.ds((s % 2) * L, L)] = src
        kbuf[pl.ds(s * L, L)] = jnp.where(keep, 1, 0).astype(jnp.int32)
        tokc[s // 2, pl.ds((s % 2) * L, L)] = (base + s * L + lanes) >> 1
        row = (s * L + lanes) >> 1
        a = plsc.load_gather(aff_v, [row, v])
        vp = plsc.load_gather(ef_v, [base + s * L + (lanes ^ 1)])
        pa = plsc.load_gather(aff_v, [row, vp])
        wn = a / jnp.maximum(a + pa, 1e-9)
        wbuf[pl.ds(s * L, L)] = jnp.where(keep, wn, 0.0)

    @pl.loop((wid + 1) * NSL, A // L)
    def _(s):
        v = ef_v[pl.ds(s * L, L)]
        plsc.addupdate_scatter(hist, [v], ones)

    hv = jnp.where(lanes < E, hist[...], jnp.int32(1 << 30))
    mn = jnp.min(hv)
    fe = plsc.all_reduce_ffs(hv == mn)
    trash = fe * C + (C - 1)

    for s in range(NSL):
        sv = dstc[s // 2, pl.ds((s % 2) * L, L)]
        kv = kbuf[pl.ds(s * L, L)]
        dstc[s // 2, pl.ds((s % 2) * L, L)] = jnp.where(kv > 0, sv, trash)

    pltpu.sync_copy(dstc, dst_hbm.at[pl.ds(wid * NCH, NCH)])
    pltpu.sync_copy(wbuf, w_hbm.at[pl.ds(base, APW)])

    rows = [rows0, rows1]
    gsem = [g0, g1]
    ssem = [s0, s1]
    gh = [None] * NCH
    sh = [None] * NCH
    gh[0] = pltpu.async_copy(x_hbm.at[tokc.at[0]], rows[0], gsem[0])
    for j in range(NCH):
        b = j % 2
        gh[j].wait()
        sh[j] = pltpu.async_copy(rows[b], buf_hbm.at[dstc.at[j]], ssem[b])
        if j + 1 < NCH:
            if j >= 1:
                sh[j - 1].wait()
            gh[j + 1] = pltpu.async_copy(
                x_hbm.at[tokc.at[j + 1]], rows[1 - b], gsem[1 - b])
    sh[NCH - 2].wait()
    sh[NCH - 1].wait()


def _routing(ef, x, aff):
    f = pl.kernel(
        _routing_body,
        out_type=[
            jax.ShapeDtypeStruct((E * C, H), jnp.float32),
            jax.ShapeDtypeStruct((NW * NCH, ROWCH), jnp.int32),
            jax.ShapeDtypeStruct((A,), jnp.float32),
        ],
        mesh=_MESH,
        compiler_params=_SC_PARAMS,
        scratch_types=[
            pltpu.VMEM((A,), jnp.int32),
            pltpu.VMEM((L,), jnp.int32),
            pltpu.VMEM((TPW, E), jnp.float32),
            pltpu.VMEM((NCH, ROWCH), jnp.int32),
            pltpu.VMEM((NCH, ROWCH), jnp.int32),
            pltpu.VMEM((APW,), jnp.int32),
            pltpu.VMEM((APW,), jnp.float32),
            pltpu.VMEM((ROWCH, H), jnp.float32),
            pltpu.VMEM((ROWCH, H), jnp.float32),
            pltpu.SemaphoreType.DMA,
            pltpu.SemaphoreType.DMA,
            pltpu.SemaphoreType.DMA,
            pltpu.SemaphoreType.DMA,
        ],
    )
    return f(ef, x, aff)


BFI = 1024
NF = I // BFI
SUBW = 256
NSUB = BFI // SUBW


def _mlp_body(a_ref, wg_ref, wu_ref, w2_ref, out_ref, abf):
    f = pl.program_id(1)

    @pl.when(f == 0)
    def _():
        abf[...] = a_ref[0].astype(jnp.bfloat16)

    ab = abf[...]
    part = None
    for k in range(NSUB):
        sl = pl.ds(k * SUBW, SUBW)
        wg = wg_ref[0, :, sl].astype(jnp.bfloat16)
        wu = wu_ref[0, :, sl].astype(jnp.bfloat16)
        w2 = w2_ref[0, sl, :].astype(jnp.bfloat16)
        gate = jnp.dot(ab, wg, preferred_element_type=jnp.float32)
        up = jnp.dot(ab, wu, preferred_element_type=jnp.float32)
        inter = (gate * jax.nn.sigmoid(gate) * up).astype(jnp.bfloat16)
        pk = jnp.dot(inter, w2, preferred_element_type=jnp.float32)
        part = pk if part is None else part + pk

    @pl.when(f == 0)
    def _():
        out_ref[0] = part

    @pl.when(f > 0)
    def _():
        out_ref[0] = out_ref[0] + part


def _mlp(buf, gate_up_w, down_w):
    return pl.pallas_call(
        _mlp_body,
        grid=(E, NF),
        in_specs=[
            pl.BlockSpec((1, C, H), lambda e, f: (e, 0, 0)),
            pl.BlockSpec((1, H, BFI), lambda e, f: (e, 0, f)),
            pl.BlockSpec((1, H, BFI), lambda e, f: (e, 0, f + NF)),
            pl.BlockSpec((1, BFI, H), lambda e, f: (e, f, 0)),
        ],
        out_specs=pl.BlockSpec((1, C, H), lambda e, f: (e, 0, 0)),
        out_shape=jax.ShapeDtypeStruct((E, C, H), jnp.float32),
        scratch_shapes=[pltpu.VMEM((C, H), jnp.bfloat16)],
        compiler_params=pltpu.CompilerParams(
            dimension_semantics=("arbitrary", "arbitrary")),
    )(buf, gate_up_w, gate_up_w, down_w)


def _combine_body(oute_hbm, dst_hbm, w_hbm, out_hbm, dstc, wv,
                  rows0, rows1, orow0, orow1, g0, g1, s0, s1):
    wid = lax.axis_index("s") * NC + lax.axis_index("c")
    base = wid * APW
    tok0 = wid * TPW

    pltpu.sync_copy(dst_hbm.at[pl.ds(wid * NCH, NCH)], dstc)
    pltpu.sync_copy(w_hbm.at[pl.ds(base, APW)], wv)

    rows = [rows0, rows1]
    orow = [orow0, orow1]
    gsem = [g0, g1]
    ssem = [s0, s1]
    gh = [None] * NCH
    sh = [None] * NCH
    gh[0] = pltpu.async_copy(oute_hbm.at[dstc.at[0]], rows[0], gsem[0])
    for j in range(NCH):
        b = j % 2
        gh[j].wait()
        if j + 1 < NCH:
            gh[j + 1] = pltpu.async_copy(
                oute_hbm.at[dstc.at[j + 1]], rows[1 - b], gsem[1 - b])
        if j >= 2:
            sh[j - 2].wait()
        for t in range(ROWCH // 2):
            li = j * ROWCH + 2 * t
            wsl = wv[pl.ds((li // L) * L, L)]
            w0 = wsl[li % L]
            w1 = wsl[li % L + 1]

            @pl.loop(0, H, step=4 * L)
            def _(h):
                for u in range(4):
                    hh = h + u * L
                    r0 = rows[b][2 * t, pl.ds(hh, L)]
                    r1 = rows[b][2 * t + 1, pl.ds(hh, L)]
                    orow[b][t, pl.ds(hh, L)] = w0 * r0 + w1 * r1

        sh[j] = pltpu.async_copy(
            orow[b], out_hbm.at[pl.ds(tok0 + j * (ROWCH // 2), ROWCH // 2)],
            ssem[b])
    sh[NCH - 2].wait()
    sh[NCH - 1].wait()


def _combine(oute, dst, w):
    f = pl.kernel(
        _combine_body,
        out_type=jax.ShapeDtypeStruct((T, H), jnp.float32),
        mesh=_MESH,
        compiler_params=_SC_PARAMS,
        scratch_types=[
            pltpu.VMEM((NCH, ROWCH), jnp.int32),
            pltpu.VMEM((APW,), jnp.float32),
            pltpu.VMEM((ROWCH, H), jnp.float32),
            pltpu.VMEM((ROWCH, H), jnp.float32),
            pltpu.VMEM((ROWCH // 2, H), jnp.float32),
            pltpu.VMEM((ROWCH // 2, H), jnp.float32),
            pltpu.SemaphoreType.DMA,
            pltpu.SemaphoreType.DMA,
            pltpu.SemaphoreType.DMA,
            pltpu.SemaphoreType.DMA,
        ],
    )
    return f(oute, dst, w)


def kernel(hidden_states, expert_affinities, expert_index, gate_up_w, down_w):
    x = hidden_states.reshape(T, H)
    ef = expert_index.reshape(A).astype(jnp.int32)
    buf, dst, w = _routing(ef, x, expert_affinities)
    oute = _mlp(buf.reshape(E, C, H), gate_up_w, down_w)
    out = _combine(oute.reshape(E * C, H), dst, w)
    return out.reshape(hidden_states.shape)


# speedup vs baseline: 1.0203x; 1.0203x over previous
import dataclasses
import functools
import math

import jax
import jax.numpy as jnp
from jax import lax
from jax.experimental import pallas as pl
from jax.experimental.pallas import tpu as pltpu
from jax.experimental.pallas import tpu_sc as plsc

E = 8
K = 2
H = 1024
I = 2048
T = 2048
CF = 2.0
C = int(math.ceil(CF * T * K / E))
A = T * K

NC, NS, L = 2, 16, 16
NW = NC * NS
APW = A // NW
TPW = T // NW
NSL = APW // L
ROWCH = 32
NCH = APW // ROWCH

_MESH = plsc.VectorSubcoreMesh(core_axis_name="c", subcore_axis_name="s",
                               num_cores=NC, num_subcores=NS)

_SC_PARAMS = pltpu.CompilerParams()
if "needs_layout_passes" in pltpu.CompilerParams.__dataclass_fields__:
    _SC_PARAMS = dataclasses.replace(_SC_PARAMS, needs_layout_passes=False)


def _routing_body(ef_hbm, x_hbm, aff_hbm, buf_hbm, dst_hbm, w_hbm,
                  ef_v, hist, aff_v, dstc, dste, dsto, kbuf, wbuf, xr0, xr1,
                  g0, g1, se0, se1, so0, so1):
    wid = lax.axis_index("s") * NC + lax.axis_index("c")
    base = wid * APW
    tok0 = wid * TPW

    pltpu.sync_copy(ef_hbm, ef_v)
    pltpu.sync_copy(aff_hbm.at[pl.ds(tok0, TPW)], aff_v)

    lanes = lax.iota(jnp.int32, L)
    ones = jnp.ones((L,), jnp.int32)
    hist[...] = jnp.zeros((L,), jnp.int32)

    @pl.loop(0, wid * NSL)
    def _(s):
        v = ef_v[pl.ds(s * L, L)]
        plsc.addupdate_scatter(hist, [v], ones)

    for s in range(NSL):
        v = ef_v[pl.ds(base + s * L, L)]
        rank = jnp.zeros((L,), jnp.int32)
        for e in range(E):
            m = v == e
            cs = plsc.cumsum(jnp.where(m, 1, 0).astype(jnp.int32))
            rank = jnp.where(m, cs - 1, rank)
        basecnt = plsc.load_gather(hist, [v])
        plsc.addupdate_scatter(hist, [v], ones)
        pos = basecnt + rank
        keep = pos < C
        posc = jnp.minimum(pos, C - 1)
        src = v * C + posc
        dstc[s // 2, pl.ds((s % 2) * L, L)] = src
        kbuf[pl.ds(s * L, L)] = jnp.where(keep, 1, 0).astype(jnp.int32)
        row = (s * L + lanes) >> 1
        a = plsc.load_gather(aff_v, [row, v])
        vp = plsc.load_gather(ef_v, [base + s * L + (lanes ^ 1)])
        pa = plsc.load_gather(aff_v, [row, vp])
        wn = a / jnp.maximum(a + pa, 1e-9)
        wbuf[pl.ds(s * L, L)] = jnp.where(keep, wn, 0.0)

    @pl.loop((wid + 1) * NSL, A // L)
    def _(s):
        v = ef_v[pl.ds(s * L, L)]
        plsc.addupdate_scatter(hist, [v], ones)

    hv = jnp.where(lanes < E, hist[...], jnp.int32(1 << 30))
    mn = jnp.min(hv)
    fe = plsc.all_reduce_ffs(hv == mn)
    trash = fe * C + (C - 1)

    for s in range(NSL):
        sv = dstc[s // 2, pl.ds((s % 2) * L, L)]
        kv = kbuf[pl.ds(s * L, L)]
        dstc[s // 2, pl.ds((s % 2) * L, L)] = jnp.where(kv > 0, sv, trash)

    for j in range(NCH):
        jv = jnp.full((L,), j, jnp.int32)
        dste[j, :] = plsc.load_gather(dstc, [jv, 2 * lanes])
        dsto[j, :] = plsc.load_gather(dstc, [jv, 2 * lanes + 1])

    pltpu.sync_copy(dstc, dst_hbm.at[pl.ds(wid * NCH, NCH)])
    pltpu.sync_copy(wbuf, w_hbm.at[pl.ds(base, APW)])

    xr = [xr0, xr1]
    gsem = [g0, g1]
    esem = [se0, se1]
    osem = [so0, so1]
    gh = [None] * NCH
    eh = [None] * NCH
    oh = [None] * NCH
    gh[0] = pltpu.async_copy(x_hbm.at[pl.ds(tok0, L)], xr[0], gsem[0])
    for j in range(NCH):
        b = j % 2
        gh[j].wait()
        eh[j] = pltpu.async_copy(xr[b], buf_hbm.at[dste.at[j]], esem[b])
        oh[j] = pltpu.async_copy(xr[b], buf_hbm.at[dsto.at[j]], osem[b])
        if j + 1 < NCH:
            if j >= 1:
                eh[j - 1].wait()
                oh[j - 1].wait()
            gh[j + 1] = pltpu.async_copy(
                x_hbm.at[pl.ds(tok0 + (j + 1) * L, L)], xr[1 - b],
                gsem[1 - b])
    for j in (NCH - 2, NCH - 1):
        eh[j].wait()
        oh[j].wait()


def _routing(ef, x, aff):
    f = pl.kernel(
        _routing_body,
        out_type=[
            jax.ShapeDtypeStruct((E * C, H), jnp.float32),
            jax.ShapeDtypeStruct((NW * NCH, ROWCH), jnp.int32),
            jax.ShapeDtypeStruct((A,), jnp.float32),
        ],
        mesh=_MESH,
        compiler_params=_SC_PARAMS,
        scratch_types=[
            pltpu.VMEM((A,), jnp.int32),
            pltpu.VMEM((L,), jnp.int32),
            pltpu.VMEM((TPW, E), jnp.float32),
            pltpu.VMEM((NCH, ROWCH), jnp.int32),
            pltpu.VMEM((NCH, L), jnp.int32),
            pltpu.VMEM((NCH, L), jnp.int32),
            pltpu.VMEM((APW,), jnp.int32),
            pltpu.VMEM((APW,), jnp.float32),
            pltpu.VMEM((L, H), jnp.float32),
            pltpu.VMEM((L, H), jnp.float32),
            pltpu.SemaphoreType.DMA,
            pltpu.SemaphoreType.DMA,
            pltpu.SemaphoreType.DMA,
            pltpu.SemaphoreType.DMA,
            pltpu.SemaphoreType.DMA,
            pltpu.SemaphoreType.DMA,
        ],
    )
    return f(ef, x, aff)


BFI = 1024
NF = I // BFI
SUBW = 256
NSUB = BFI // SUBW


def _mlp_body(a_ref, wg_ref, wu_ref, w2_ref, out_ref, abf):
    f = pl.program_id(1)

    @pl.when(f == 0)
    def _():
        abf[...] = a_ref[0].astype(jnp.bfloat16)

    ab = abf[...]
    part = None
    for k in range(NSUB):
        sl = pl.ds(k * SUBW, SUBW)
        wg = wg_ref[0, :, sl].astype(jnp.bfloat16)
        wu = wu_ref[0, :, sl].astype(jnp.bfloat16)
        w2 = w2_ref[0, sl, :].astype(jnp.bfloat16)
        gate = jnp.dot(ab, wg, preferred_element_type=jnp.float32)
        up = jnp.dot(ab, wu, preferred_element_type=jnp.float32)
        inter = (gate * jax.nn.sigmoid(gate) * up).astype(jnp.bfloat16)
        pk = jnp.dot(inter, w2, preferred_element_type=jnp.float32)
        part = pk if part is None else part + pk

    @pl.when(f == 0)
    def _():
        out_ref[0] = part

    @pl.when(f > 0)
    def _():
        out_ref[0] = out_ref[0] + part


def _mlp(buf, gate_up_w, down_w):
    return pl.pallas_call(
        _mlp_body,
        grid=(E, NF),
        in_specs=[
            pl.BlockSpec((1, C, H), lambda e, f: (e, 0, 0)),
            pl.BlockSpec((1, H, BFI), lambda e, f: (e, 0, f)),
            pl.BlockSpec((1, H, BFI), lambda e, f: (e, 0, f + NF)),
            pl.BlockSpec((1, BFI, H), lambda e, f: (e, f, 0)),
        ],
        out_specs=pl.BlockSpec((1, C, H), lambda e, f: (e, 0, 0)),
        out_shape=jax.ShapeDtypeStruct((E, C, H), jnp.float32),
        scratch_shapes=[pltpu.VMEM((C, H), jnp.bfloat16)],
        compiler_params=pltpu.CompilerParams(
            dimension_semantics=("arbitrary", "arbitrary")),
    )(buf, gate_up_w, gate_up_w, down_w)


def _combine_body(oute_hbm, dst_hbm, w_hbm, out_hbm, dstc, wv,
                  rows0, rows1, orow0, orow1, g0, g1, s0, s1):
    wid = lax.axis_index("s") * NC + lax.axis_index("c")
    base = wid * APW
    tok0 = wid * TPW

    pltpu.sync_copy(dst_hbm.at[pl.ds(wid * NCH, NCH)], dstc)
    pltpu.sync_copy(w_hbm.at[pl.ds(base, APW)], wv)

    rows = [rows0, rows1]
    orow = [orow0, orow1]
    gsem = [g0, g1]
    ssem = [s0, s1]
    gh = [None] * NCH
    sh = [None] * NCH
    gh[0] = pltpu.async_copy(oute_hbm.at[dstc.at[0]], rows[0], gsem[0])
    for j in range(NCH):
        b = j % 2
        gh[j].wait()
        if j + 1 < NCH:
            gh[j + 1] = pltpu.async_copy(
                oute_hbm.at[dstc.at[j + 1]], rows[1 - b], gsem[1 - b])
        if j >= 2:
            sh[j - 2].wait()
        for t in range(ROWCH // 2):
            li = j * ROWCH + 2 * t
            wsl = wv[pl.ds((li // L) * L, L)]
            w0 = wsl[li % L]
            w1 = wsl[li % L + 1]

            @pl.loop(0, H, step=4 * L)
            def _(h):
                for u in range(4):
                    hh = h + u * L
                    r0 = rows[b][2 * t, pl.ds(hh, L)]
                    r1 = rows[b][2 * t + 1, pl.ds(hh, L)]
                    orow[b][t, pl.ds(hh, L)] = w0 * r0 + w1 * r1

        sh[j] = pltpu.async_copy(
            orow[b], out_hbm.at[pl.ds(tok0 + j * (ROWCH // 2), ROWCH // 2)],
            ssem[b])
    sh[NCH - 2].wait()
    sh[NCH - 1].wait()


def _combine(oute, dst, w):
    f = pl.kernel(
        _combine_body,
        out_type=jax.ShapeDtypeStruct((T, H), jnp.float32),
        mesh=_MESH,
        compiler_params=_SC_PARAMS,
        scratch_types=[
            pltpu.VMEM((NCH, ROWCH), jnp.int32),
            pltpu.VMEM((APW,), jnp.float32),
            pltpu.VMEM((ROWCH, H), jnp.float32),
            pltpu.VMEM((ROWCH, H), jnp.float32),
            pltpu.VMEM((ROWCH // 2, H), jnp.float32),
            pltpu.VMEM((ROWCH // 2, H), jnp.float32),
            pltpu.SemaphoreType.DMA,
            pltpu.SemaphoreType.DMA,
            pltpu.SemaphoreType.DMA,
            pltpu.SemaphoreType.DMA,
        ],
    )
    return f(oute, dst, w)


def kernel(hidden_states, expert_affinities, expert_index, gate_up_w, down_w):
    x = hidden_states.reshape(T, H)
    ef = expert_index.reshape(A).astype(jnp.int32)
    buf, dst, w = _routing(ef, x, expert_affinities)
    oute = _mlp(buf.reshape(E, C, H), gate_up_w, down_w)
    out = _combine(oute.reshape(E * C, H), dst, w)
    return out.reshape(hidden_states.shape)
